# Initial kernel scaffold; baseline (speedup 1.0000x reference)
import jax
import jax.numpy as jnp
from jax.experimental import pallas as pl


def kernel(src, index, dim, dim_size):
    raise NotImplementedError("write your pallas kernel here")


# speedup vs baseline: 2.0812x; 2.0812x over previous
import jax
import jax.numpy as jnp
from jax import lax
from jax.experimental import pallas as pl
from jax.experimental.pallas import tpu as pltpu
from jax.experimental.pallas import tpu_sc as plsc

N_EDGES = 320000
N_NODES = 10000
D_FEAT = 128

NUM_CORES = 2
NUM_SUBCORES = 16
NODES_PER_CORE = N_NODES // NUM_CORES
ACC_ROWS = 5008
DUMMY_ROW = 5000
E_PER_TILE = N_EDGES // NUM_SUBCORES
CHUNK = 80
N_CHUNKS = E_PER_TILE // CHUNK
R_PER_TILE = 312
R_TAIL = NODES_PER_CORE - NUM_SUBCORES * R_PER_TILE
Z_TAIL = ACC_ROWS - NUM_SUBCORES * R_PER_TILE


def _sc_body(src_hbm, idx_hbm, out_hbm, stage, rows, idxb, idxb2, acc):
    c = lax.axis_index("c")
    s = lax.axis_index("s")
    lo = c * NODES_PER_CORE
    hi = lo + NODES_PER_CORE

    def zero_row(r, carry):
        for j in range(D_FEAT // 16):
            stage[r, pl.ds(j * 16, 16)] = jnp.zeros((16,), jnp.float32)
        return carry

    lax.fori_loop(0, R_PER_TILE, zero_row, 0)
    pltpu.sync_copy(stage, acc.at[pl.ds(s * R_PER_TILE, R_PER_TILE)])

    @pl.when(s == 0)
    def _():
        pltpu.sync_copy(
            stage.at[pl.ds(0, Z_TAIL)],
            acc.at[pl.ds(NUM_SUBCORES * R_PER_TILE, Z_TAIL)],
        )

    plsc.subcore_barrier()

    ebase = s * E_PER_TILE

    def chunk_step(i, carry):
        b = ebase + i * CHUNK
        pltpu.sync_copy(idx_hbm.at[pl.ds(b, CHUNK)], idxb)
        first = idxb[pl.ds(0, 16)][0]
        last = idxb[pl.ds(CHUNK - 16, 16)][15]

        @pl.when(jnp.logical_and(last >= lo, first < hi))
        def _():
            pltpu.sync_copy(src_hbm.at[pl.ds(b, CHUNK)], rows)
            for j in range(CHUNK // 16):
                v = idxb[pl.ds(j * 16, 16)]
                ok = jnp.logical_and(v >= lo, v < hi)
                idxb2[pl.ds(j * 16, 16)] = jnp.where(ok, v - lo, DUMMY_ROW)
            pltpu.sync_copy(rows, acc.at[idxb2], add=True)

        return carry

    lax.fori_loop(0, N_CHUNKS, chunk_step, 0)
    plsc.subcore_barrier()

    rbase = s * R_PER_TILE
    pltpu.sync_copy(acc.at[pl.ds(rbase, R_PER_TILE)], stage)
    pltpu.sync_copy(stage, out_hbm.at[pl.ds(lo + rbase, R_PER_TILE)])

    @pl.when(s == 0)
    def _():
        tbase = NUM_SUBCORES * R_PER_TILE
        pltpu.sync_copy(acc.at[pl.ds(tbase, R_TAIL)], stage.at[pl.ds(0, R_TAIL)])
        pltpu.sync_copy(
            stage.at[pl.ds(0, R_TAIL)], out_hbm.at[pl.ds(lo + tbase, R_TAIL)]
        )


@jax.jit
def _segment_sum_sc(src, index):
    mesh = plsc.VectorSubcoreMesh(core_axis_name="c", subcore_axis_name="s")
    return pl.kernel(
        _sc_body,
        out_type=jax.ShapeDtypeStruct((N_NODES, D_FEAT), jnp.float32),
        mesh=mesh,
        scratch_types=[
            pltpu.VMEM((R_PER_TILE, D_FEAT), jnp.float32),
            pltpu.VMEM((CHUNK, D_FEAT), jnp.float32),
            pltpu.VMEM((CHUNK,), jnp.int32),
            pltpu.VMEM((CHUNK,), jnp.int32),
            pltpu.VMEM_SHARED((ACC_ROWS, D_FEAT), jnp.float32),
        ],
    )(src, index)


def kernel(src, index, dim, dim_size):
    del dim, dim_size
    return _segment_sum_sc(src, index.astype(jnp.int32))


# speedup vs baseline: 6.7792x; 3.2574x over previous
import jax
import jax.numpy as jnp
from jax import lax
from jax.experimental import pallas as pl
from jax.experimental.pallas import tpu as pltpu
from jax.experimental.pallas import tpu_sc as plsc

N_EDGES = 320000
N_NODES = 10000
D_FEAT = 128

NUM_CORES = 2
NUM_SUBCORES = 16
NODES_PER_CORE = N_NODES // NUM_CORES
ACC_ROWS = 5008
DUMMY_ROW = 5000
E_PER_TILE = N_EDGES // NUM_SUBCORES
CHUNK = 80
N_CHUNKS = E_PER_TILE // CHUNK
NBUF = 4
R_PER_TILE = 312
R_TAIL = NODES_PER_CORE - NUM_SUBCORES * R_PER_TILE
Z_PER_TILE = ACC_ROWS // NUM_SUBCORES
WB_OFFS = (0, 80, 160, 232)
Z_OFFS = (0, 80, 160, 233)


def _sc_body(src_hbm, idx_hbm, out_hbm, stage, idxall,
             rows0, rows1, rows2, rows3,
             li0, li1, li2, li3,
             lsem0, lsem1, lsem2, lsem3,
             ssem0, ssem1, ssem2, ssem3,
             acc):
    rows = (rows0, rows1, rows2, rows3)
    lidx = (li0, li1, li2, li3)
    lsem = (lsem0, lsem1, lsem2, lsem3)
    ssem = (ssem0, ssem1, ssem2, ssem3)
    c = lax.axis_index("c")
    s = lax.axis_index("s")
    lo = c * NODES_PER_CORE
    hi = lo + NODES_PER_CORE

    def zero_row(r, carry):
        for j in range(D_FEAT // 16):
            stage[r, pl.ds(j * 16, 16)] = jnp.zeros((16,), jnp.float32)
        return carry

    lax.fori_loop(0, CHUNK, zero_row, 0)
    zbase = s * Z_PER_TILE
    for off in Z_OFFS:
        pltpu.sync_copy(stage, acc.at[pl.ds(zbase + off, CHUNK)])

    pltpu.sync_copy(idx_hbm.at[pl.ds(s * E_PER_TILE, E_PER_TILE)], idxall)
    plsc.subcore_barrier()

    def first_chunk_where(pred):
        def step(t, st):
            lo_c, hi_c = st
            mid = (lo_c + hi_c) // 2
            v = pred(mid)
            new_hi = jnp.where(v, mid, hi_c)
            new_lo = jnp.where(v, lo_c, mid + 1)
            done = lo_c >= hi_c
            return (jnp.where(done, lo_c, new_lo),
                    jnp.where(done, hi_c, new_hi))

        return lax.fori_loop(0, 8, step, (0, N_CHUNKS))[0]

    c_start = first_chunk_where(
        lambda ci: idxall[pl.ds(ci * CHUNK + CHUNK - 16, 16)][15] >= lo)
    c_end = first_chunk_where(
        lambda ci: idxall[pl.ds(ci * CHUNK, 16)][0] >= hi)

    ebase = s * E_PER_TILE

    def load_slice(i):
        return src_hbm.at[pl.ds(ebase + i * CHUNK, CHUNK)]

    for b in range(NBUF):
        @pl.when(c_start + b < c_end)
        def _(b=b):
            pltpu.async_copy(load_slice(c_start + b), rows[b], lsem[b])

    n_groups = (c_end - c_start + NBUF - 1) // NBUF

    def group(g, carry):
        i0 = c_start + g * NBUF
        for b in range(NBUF):
            i = i0 + b

            @pl.when(i < c_end)
            def _(b=b, i=i):
                pltpu.make_async_copy(load_slice(i), rows[b], lsem[b]).wait()
                for j in range(CHUNK // 16):
                    v = idxall[pl.ds(i * CHUNK + j * 16, 16)]
                    ok = jnp.logical_and(v >= lo, v < hi)
                    lidx[b][pl.ds(j * 16, 16)] = jnp.where(ok, v - lo, DUMMY_ROW)
                pltpu.async_copy(rows[b], acc.at[lidx[b]], ssem[b], add=True)
                pltpu.make_async_copy(rows[b], acc.at[lidx[b]], ssem[b]).wait()

                @pl.when(i + NBUF < c_end)
                def _():
                    pltpu.async_copy(load_slice(i + NBUF), rows[b], lsem[b])

        return carry

    lax.fori_loop(0, n_groups, group, 0)
    plsc.subcore_barrier()

    rbase = s * R_PER_TILE
    for b, off in enumerate(WB_OFFS):
        pltpu.async_copy(acc.at[pl.ds(rbase + off, CHUNK)], rows[b], lsem[b])
    for b, off in enumerate(WB_OFFS):
        pltpu.make_async_copy(
            acc.at[pl.ds(rbase + off, CHUNK)], rows[b], lsem[b]).wait()
        pltpu.async_copy(rows[b], out_hbm.at[pl.ds(lo + rbase + off, CHUNK)],
                         ssem[b])
    for b, off in enumerate(WB_OFFS):
        pltpu.make_async_copy(
            rows[b], out_hbm.at[pl.ds(lo + rbase + off, CHUNK)], ssem[b]).wait()

    @pl.when(s == 0)
    def _():
        tbase = NUM_SUBCORES * R_PER_TILE
        pltpu.sync_copy(acc.at[pl.ds(tbase, R_TAIL)], stage.at[pl.ds(0, R_TAIL)])
        pltpu.sync_copy(
            stage.at[pl.ds(0, R_TAIL)], out_hbm.at[pl.ds(lo + tbase, R_TAIL)]
        )


@jax.jit
def _segment_sum_sc(src, index):
    mesh = plsc.VectorSubcoreMesh(core_axis_name="c", subcore_axis_name="s")
    return pl.kernel(
        _sc_body,
        out_type=jax.ShapeDtypeStruct((N_NODES, D_FEAT), jnp.float32),
        mesh=mesh,
        scratch_types=[
            pltpu.VMEM((CHUNK, D_FEAT), jnp.float32),
            pltpu.VMEM((E_PER_TILE,), jnp.int32),
            pltpu.VMEM((CHUNK, D_FEAT), jnp.float32),
            pltpu.VMEM((CHUNK, D_FEAT), jnp.float32),
            pltpu.VMEM((CHUNK, D_FEAT), jnp.float32),
            pltpu.VMEM((CHUNK, D_FEAT), jnp.float32),
            pltpu.VMEM((CHUNK,), jnp.int32),
            pltpu.VMEM((CHUNK,), jnp.int32),
            pltpu.VMEM((CHUNK,), jnp.int32),
            pltpu.VMEM((CHUNK,), jnp.int32),
            pltpu.SemaphoreType.DMA,
            pltpu.SemaphoreType.DMA,
            pltpu.SemaphoreType.DMA,
            pltpu.SemaphoreType.DMA,
            pltpu.SemaphoreType.DMA,
            pltpu.SemaphoreType.DMA,
            pltpu.SemaphoreType.DMA,
            pltpu.SemaphoreType.DMA,
            pltpu.VMEM_SHARED((ACC_ROWS, D_FEAT), jnp.float32),
        ],
    )(src, index)


def kernel(src, index, dim, dim_size):
    del dim, dim_size
    return _segment_sum_sc(src, index.astype(jnp.int32))
